# baseline (speedup 1.0000x reference)
import functools

import jax
import jax.numpy as jnp
from jax import lax
from jax.experimental import pallas as pl
from jax.experimental.pallas import tpu as pltpu
from jax.experimental.pallas import tpu_sc as plsc

N_NODE = 10000
N_EDGE = 2000
N_PAIR = 160000
D = 128

NC = 2
NS = 16
ROWS_CORE = N_EDGE // NC
ROWS_P0 = 512
ROWS_P1 = ROWS_CORE - ROWS_P0
NBLK = -(-N_EDGE // 128)
UNIT = 8 * NBLK * 128
UNITS_P0 = ROWS_P0 // 8
UNITS_P1 = ROWS_P1 // 8
P0_WORDS = UNITS_P0 * UNIT
P1_WORDS = UNITS_P1 * UNIT
DUMP = P0_WORDS
PAIRS_MAIN = (N_PAIR // 128 // NS) * 128
TAIL_BASE = PAIRS_MAIN * NS
IDX_ROWS = PAIRS_MAIN // 128 + 1
ZCHUNK = 4096


def _mask_body(p0_hbm, p1_hbm, out_hbm, shared, pairv, tailv, idxv,
               onesv, zbuf, semp, semz, sems, semo):
    cid = lax.axis_index("c")
    sid = lax.axis_index("s")

    hp0 = pltpu.async_copy(
        p0_hbm.at[pl.ds(sid * PAIRS_MAIN, PAIRS_MAIN)], pairv.at[0], semp)
    hp1 = pltpu.async_copy(
        p1_hbm.at[pl.ds(sid * PAIRS_MAIN, PAIRS_MAIN)], pairv.at[1], semp)

    def _fill_z(i, _):
        zbuf[pl.ds(i * 16, 16)] = jnp.zeros((16,), jnp.float32)
        return 0

    with jax.named_scope("ph_fill"):
        lax.fori_loop(0, ZCHUNK // 16, _fill_z, 0)
        for k in range(8):
            onesv[pl.ds(k * 16, 16)] = jnp.ones((16,), jnp.float32)

    def _fire_zero(words):
        hs = []
        base = sid * words
        nfull, rem = divmod(words, ZCHUNK)
        for c in range(nfull):
            hs.append(pltpu.async_copy(
                zbuf, shared.at[pl.ds(base + c * ZCHUNK, ZCHUNK)], semz))
        if rem:
            hs.append(pltpu.async_copy(
                zbuf.at[pl.ds(0, rem)],
                shared.at[pl.ds(base + nfull * ZCHUNK, rem)], semz))
        return hs

    zh = _fire_zero(P0_WORDS // NS)

    with jax.named_scope("ph_pairwait"):
        hp0.wait()
        hp1.wait()

    @pl.when(sid < 2)
    def _():
        pltpu.sync_copy(p0_hbm.at[pl.ds(TAIL_BASE + sid * 128, 128)],
                        tailv.at[0])
        pltpu.sync_copy(p1_hbm.at[pl.ds(TAIL_BASE + sid * 128, 128)],
                        tailv.at[1])

    row_lo = cid * ROWS_CORE

    def _compute_idx(pass_lo, pass_rows):
        def _offs(p0, p1):
            r = p0 - pass_lo
            woff = (r << 11) + p1
            return jnp.where((r >= 0) & (r < pass_rows), woff, DUMP + p1)

        def _body(i, _):
            p0 = pairv[0, pl.ds(i * 16, 16)]
            p1 = pairv[1, pl.ds(i * 16, 16)]
            idxv[i // 8, pl.ds((i % 8) * 16, 16)] = _offs(p0, p1)
            return 0

        with jax.named_scope("ph_offsets"):
            lax.fori_loop(0, PAIRS_MAIN // 16, _body, 0)
            dump = jnp.full((16,), DUMP, jnp.int32)
            for k in range(8):
                idxv[IDX_ROWS - 1, pl.ds(k * 16, 16)] = dump

            @pl.when(sid < 2)
            def _():
                for k in range(8):
                    p0 = tailv[0, pl.ds(k * 16, 16)]
                    p1 = tailv[1, pl.ds(k * 16, 16)]
                    idxv[IDX_ROWS - 1, pl.ds(k * 16, 16)] = _offs(p0, p1)

    def _scatter():
        with jax.named_scope("ph_scatter"):
            sh = [pltpu.async_copy(onesv, shared.at[idxv.at[j]], sems)
                  for j in range(IDX_ROWS)]
            for h in sh:
                h.wait()

    def _copy_out(units, unit0):
        with jax.named_scope("ph_copyout"):
            return [pltpu.async_copy(
                shared.at[pl.ds(u * UNIT, UNIT)],
                out_hbm.at[pl.ds((cid * (ROWS_CORE // 8) + unit0 + u) * UNIT,
                                 UNIT)], semo)
                for u in units]

    _compute_idx(row_lo, ROWS_P0)
    with jax.named_scope("ph_zerowait"):
        for h in zh:
            h.wait()
    plsc.subcore_barrier()
    _scatter()
    plsc.subcore_barrier()
    upt = UNITS_P0 // NS
    pend = _copy_out([sid * upt + k for k in range(upt)], 0)
    for h in pend:
        h.wait()
    plsc.subcore_barrier()

    zh = _fire_zero(P1_WORDS // NS)
    _compute_idx(row_lo + ROWS_P0, ROWS_P1)
    with jax.named_scope("ph_zerowait"):
        for h in zh:
            h.wait()
    plsc.subcore_barrier()
    _scatter()
    plsc.subcore_barrier()
    pend = _copy_out([sid + NS * k for k in range(UNITS_P1 // NS)], UNITS_P0)
    for h in pend:
        h.wait()

    @pl.when(sid < UNITS_P1 % NS)
    def _():
        u = sid + NS * (UNITS_P1 // NS)
        pltpu.sync_copy(
            shared.at[pl.ds(u * UNIT, UNIT)],
            out_hbm.at[pl.ds((cid * (ROWS_CORE // 8) + UNITS_P0 + u) * UNIT,
                             UNIT)])


_build_mask = functools.partial(
    pl.kernel,
    mesh=plsc.VectorSubcoreMesh(core_axis_name="c", subcore_axis_name="s"),
    out_type=jax.ShapeDtypeStruct(((N_EDGE // 8) * UNIT,), jnp.float32),
    scratch_types=[
        pltpu.VMEM_SHARED((P0_WORDS + 2048,), jnp.float32),
        pltpu.VMEM((2, PAIRS_MAIN), jnp.int32),
        pltpu.VMEM((2, 128), jnp.int32),
        pltpu.VMEM((IDX_ROWS, 128), jnp.int32),
        pltpu.VMEM((128,), jnp.float32),
        pltpu.VMEM((ZCHUNK,), jnp.float32),
        pltpu.SemaphoreType.DMA,
        pltpu.SemaphoreType.DMA,
        pltpu.SemaphoreType.DMA,
        pltpu.SemaphoreType.DMA,
    ],
)(_mask_body)


def _proj_body(x_ref, xe_ref, wt_ref, a_ref, xep_ref, xa_ref, xph_ref, ts_ref):
    wt = wt_ref[...]
    xp = jnp.dot(x_ref[...], wt, preferred_element_type=jnp.float32)
    xep_ref[...] = jnp.dot(xe_ref[...], wt, preferred_element_type=jnp.float32)
    xph = xp[:N_EDGE]
    xph_ref[...] = xph
    xa_ref[...] = xph * a_ref[...]
    ts_ref[...] = jnp.sum(xp[N_EDGE:], axis=0, keepdims=True)


_project = pl.pallas_call(
    _proj_body,
    out_shape=[
        jax.ShapeDtypeStruct((N_EDGE, D), jnp.float32),
        jax.ShapeDtypeStruct((N_EDGE, D), jnp.float32),
        jax.ShapeDtypeStruct((N_EDGE, D), jnp.float32),
        jax.ShapeDtypeStruct((1, D), jnp.float32),
    ],
)


def _attn_body(xep_ref, xa_ref, xph_ref, ts_ref, m_ref, xout_ref, xeout_ref):
    xep = xep_ref[...]
    s = lax.dot_general(xep, xa_ref[...], (((1,), (1,)), ((), ())),
                        preferred_element_type=jnp.float32)
    g = 1e-10 + jnp.where(m_ref[:, :N_EDGE] > 0, jnp.exp(s), 0.0)

    ze = jnp.sum(g, axis=1, keepdims=True) + (N_NODE - N_EDGE) * 1e-10
    pe = g / ze
    xe_out = (jnp.dot(pe, xph_ref[...], preferred_element_type=jnp.float32)
              + (1e-10 / ze) * ts_ref[...])
    xeout_ref[...] = jnp.where(xe_out > 0, xe_out, jnp.exp(xe_out) - 1.0)

    zn = jnp.sum(g, axis=0, keepdims=True)
    qn = g / zn
    x_head = lax.dot_general(qn, xep, (((0,), (0,)), ((), ())),
                             preferred_element_type=jnp.float32)
    xout_ref[:N_EDGE] = jnp.where(x_head > 0, x_head, jnp.exp(x_head) - 1.0)
    x_tail = jnp.sum(xep, axis=0, keepdims=True) * (1.0 / N_EDGE)
    x_tail = jnp.where(x_tail > 0, x_tail, jnp.exp(x_tail) - 1.0)
    xout_ref[N_EDGE:] = jnp.broadcast_to(x_tail, (N_NODE - N_EDGE, D))


_attend = pl.pallas_call(
    _attn_body,
    out_shape=[
        jax.ShapeDtypeStruct((N_NODE, D), jnp.float32),
        jax.ShapeDtypeStruct((N_EDGE, D), jnp.float32),
    ],
)


def kernel(x, xe, pair, a, wt):
    m = _build_mask(pair[0], pair[1]).reshape(N_EDGE, 16 * 128)
    xep, xa, xph, ts = _project(x, xe, wt, a.reshape(1, D))
    x_out, xe_out = _attend(xep, xa, xph, ts, m)
    return x_out, xe_out


# speedup vs baseline: 1.0573x; 1.0573x over previous
import functools

import jax
import jax.numpy as jnp
from jax import lax
from jax.experimental import pallas as pl
from jax.experimental.pallas import tpu as pltpu
from jax.experimental.pallas import tpu_sc as plsc

N_NODE = 10000
N_EDGE = 2000
N_PAIR = 160000
D = 128

NC = 2
NS = 16
ROWS_CORE = N_EDGE // NC
ROWS_P0 = 512
ROWS_P1 = ROWS_CORE - ROWS_P0
NBLK = -(-N_EDGE // 128)
UNIT = 8 * NBLK * 128
UNITS_P0 = ROWS_P0 // 8
UNITS_P1 = ROWS_P1 // 8
P0_WORDS = UNITS_P0 * UNIT
P1_WORDS = UNITS_P1 * UNIT
DUMP = P0_WORDS
PAIRS_MAIN = (N_PAIR // 128 // NS) * 128
TAIL_BASE = PAIRS_MAIN * NS
IDX_ROWS = PAIRS_MAIN // 128 + 1
ZCHUNK = 4096


def _mask_body(p0_hbm, p1_hbm, out_hbm, shared, pairv, tailv, idxv,
               onesv, zbuf, semp, semz, sems, semo):
    cid = lax.axis_index("c")
    sid = lax.axis_index("s")

    hp0 = pltpu.async_copy(
        p0_hbm.at[pl.ds(sid * PAIRS_MAIN, PAIRS_MAIN)], pairv.at[0], semp)
    hp1 = pltpu.async_copy(
        p1_hbm.at[pl.ds(sid * PAIRS_MAIN, PAIRS_MAIN)], pairv.at[1], semp)

    def _fill_z(i, _):
        zbuf[pl.ds(i * 16, 16)] = jnp.zeros((16,), jnp.float32)
        return 0

    with jax.named_scope("ph_fill"):
        lax.fori_loop(0, ZCHUNK // 16, _fill_z, 0)
        for k in range(8):
            onesv[pl.ds(k * 16, 16)] = jnp.ones((16,), jnp.float32)

    def _fire_zero(words):
        hs = []
        base = sid * words
        nfull, rem = divmod(words, ZCHUNK)
        for c in range(nfull):
            hs.append(pltpu.async_copy(
                zbuf, shared.at[pl.ds(base + c * ZCHUNK, ZCHUNK)], semz))
        if rem:
            hs.append(pltpu.async_copy(
                zbuf.at[pl.ds(0, rem)],
                shared.at[pl.ds(base + nfull * ZCHUNK, rem)], semz))
        return hs

    zh = _fire_zero(P0_WORDS // NS)

    with jax.named_scope("ph_pairwait"):
        hp0.wait()
        hp1.wait()

    @pl.when(sid < 2)
    def _():
        pltpu.sync_copy(p0_hbm.at[pl.ds(TAIL_BASE + sid * 128, 128)],
                        tailv.at[0])
        pltpu.sync_copy(p1_hbm.at[pl.ds(TAIL_BASE + sid * 128, 128)],
                        tailv.at[1])

    row_lo = cid * ROWS_CORE

    def _compute_idx(pass_lo, pass_rows):
        def _offs(p0, p1):
            r = p0 - pass_lo
            woff = (r << 11) + p1
            return jnp.where((r >= 0) & (r < pass_rows), woff, DUMP + p1)

        def _body(i, _):
            p0 = pairv[0, pl.ds(i * 16, 16)]
            p1 = pairv[1, pl.ds(i * 16, 16)]
            idxv[i // 8, pl.ds((i % 8) * 16, 16)] = _offs(p0, p1)
            return 0

        with jax.named_scope("ph_offsets"):
            lax.fori_loop(0, PAIRS_MAIN // 16, _body, 0)
            dump = jnp.full((16,), DUMP, jnp.int32)
            for k in range(8):
                idxv[IDX_ROWS - 1, pl.ds(k * 16, 16)] = dump

            @pl.when(sid < 2)
            def _():
                for k in range(8):
                    p0 = tailv[0, pl.ds(k * 16, 16)]
                    p1 = tailv[1, pl.ds(k * 16, 16)]
                    idxv[IDX_ROWS - 1, pl.ds(k * 16, 16)] = _offs(p0, p1)

    def _scatter():
        with jax.named_scope("ph_scatter"):
            sh = [pltpu.async_copy(onesv, shared.at[idxv.at[j]], sems)
                  for j in range(IDX_ROWS)]
            for h in sh:
                h.wait()

    def _copy_out(units, unit0):
        with jax.named_scope("ph_copyout"):
            return [pltpu.async_copy(
                shared.at[pl.ds(u * UNIT, UNIT)],
                out_hbm.at[pl.ds((cid * (ROWS_CORE // 8) + unit0 + u) * UNIT,
                                 UNIT)], semo)
                for u in units]

    _compute_idx(row_lo, ROWS_P0)
    with jax.named_scope("ph_zerowait"):
        for h in zh:
            h.wait()
    plsc.subcore_barrier()
    _scatter()
    plsc.subcore_barrier()
    upt = UNITS_P0 // NS
    pend = _copy_out([sid * upt + k for k in range(upt)], 0)
    for h in pend:
        h.wait()
    plsc.subcore_barrier()

    zh = _fire_zero(P1_WORDS // NS)
    _compute_idx(row_lo + ROWS_P0, ROWS_P1)
    with jax.named_scope("ph_zerowait"):
        for h in zh:
            h.wait()
    plsc.subcore_barrier()
    _scatter()
    plsc.subcore_barrier()
    pend = _copy_out([sid + NS * k for k in range(UNITS_P1 // NS)], UNITS_P0)
    for h in pend:
        h.wait()

    @pl.when(sid < UNITS_P1 % NS)
    def _():
        u = sid + NS * (UNITS_P1 // NS)
        pltpu.sync_copy(
            shared.at[pl.ds(u * UNIT, UNIT)],
            out_hbm.at[pl.ds((cid * (ROWS_CORE // 8) + UNITS_P0 + u) * UNIT,
                             UNIT)])


_build_mask = functools.partial(
    pl.kernel,
    mesh=plsc.VectorSubcoreMesh(core_axis_name="c", subcore_axis_name="s"),
    out_type=jax.ShapeDtypeStruct(((N_EDGE // 8) * UNIT,), jnp.float32),
    scratch_types=[
        pltpu.VMEM_SHARED((P0_WORDS + 2048,), jnp.float32),
        pltpu.VMEM((2, PAIRS_MAIN), jnp.int32),
        pltpu.VMEM((2, 128), jnp.int32),
        pltpu.VMEM((IDX_ROWS, 128), jnp.int32),
        pltpu.VMEM((128,), jnp.float32),
        pltpu.VMEM((ZCHUNK,), jnp.float32),
        pltpu.SemaphoreType.DMA,
        pltpu.SemaphoreType.DMA,
        pltpu.SemaphoreType.DMA,
        pltpu.SemaphoreType.DMA,
    ],
)(_mask_body)


def _proj_body(x_ref, xe_ref, wt_ref, a_ref, xep_ref, xa_ref, xph_ref, ts_ref):
    wt = wt_ref[...]
    xp = jnp.dot(x_ref[...], wt, preferred_element_type=jnp.float32)
    xep_ref[...] = jnp.dot(xe_ref[...], wt, preferred_element_type=jnp.float32)
    xph = xp[:N_EDGE]
    xph_ref[...] = xph
    xa_ref[...] = xph * a_ref[...]
    ts_ref[...] = jnp.sum(xp[N_EDGE:], axis=0, keepdims=True)


_project = pl.pallas_call(
    _proj_body,
    out_shape=[
        jax.ShapeDtypeStruct((N_EDGE, D), jnp.float32),
        jax.ShapeDtypeStruct((N_EDGE, D), jnp.float32),
        jax.ShapeDtypeStruct((N_EDGE, D), jnp.float32),
        jax.ShapeDtypeStruct((1, D), jnp.float32),
    ],
)


BR = 400
NSTEPS = N_EDGE // BR


def _attn_body(xep_ref, xa_ref, xph_ref, ts_ref, m_ref,
               xout_ref, xeout_ref, nacc, zacc, xsacc):
    i = pl.program_id(0)
    xep = xep_ref[...]
    s = lax.dot_general(xep, xa_ref[...], (((1,), (1,)), ((), ())),
                        preferred_element_type=jnp.float32)
    m = jnp.reshape(m_ref[...], (BR, 16 * 128))[:, :N_EDGE]
    g = 1e-10 + jnp.where(m > 0, jnp.exp(s), 0.0)

    ze = jnp.sum(g, axis=1, keepdims=True) + (N_NODE - N_EDGE) * 1e-10
    pe = g / ze
    xe_out = (jnp.dot(pe, xph_ref[...], preferred_element_type=jnp.float32)
              + (1e-10 / ze) * ts_ref[...])
    xeout_ref[...] = jnp.where(xe_out > 0, xe_out, jnp.exp(xe_out) - 1.0)

    @pl.when(i == 0)
    def _():
        nacc[...] = jnp.zeros((N_EDGE, D), jnp.float32)
        zacc[...] = jnp.zeros((N_EDGE, 1), jnp.float32)
        xsacc[...] = jnp.zeros((1, D), jnp.float32)

    nacc[...] += lax.dot_general(g, xep, (((0,), (0,)), ((), ())),
                                 preferred_element_type=jnp.float32)
    zacc[...] += lax.dot_general(g, jnp.ones((BR, 1), jnp.float32),
                                 (((0,), (0,)), ((), ())),
                                 preferred_element_type=jnp.float32)
    xsacc[...] += jnp.sum(xep, axis=0, keepdims=True)

    @pl.when(i == NSTEPS - 1)
    def _():
        x_head = nacc[...] / zacc[...]
        xout_ref[:N_EDGE] = jnp.where(x_head > 0, x_head,
                                      jnp.exp(x_head) - 1.0)
        x_tail = xsacc[...] * (1.0 / N_EDGE)
        x_tail = jnp.where(x_tail > 0, x_tail, jnp.exp(x_tail) - 1.0)
        xout_ref[N_EDGE:] = jnp.broadcast_to(x_tail, (N_NODE - N_EDGE, D))


_attend = pl.pallas_call(
    _attn_body,
    grid=(NSTEPS,),
    in_specs=[
        pl.BlockSpec((BR, D), lambda i: (i, 0)),
        pl.BlockSpec((N_EDGE, D), lambda i: (0, 0)),
        pl.BlockSpec((N_EDGE, D), lambda i: (0, 0)),
        pl.BlockSpec((1, D), lambda i: (0, 0)),
        pl.BlockSpec((BR * 16 * 128,), lambda i: (i,)),
    ],
    out_specs=[
        pl.BlockSpec((N_NODE, D), lambda i: (0, 0)),
        pl.BlockSpec((BR, D), lambda i: (i, 0)),
    ],
    out_shape=[
        jax.ShapeDtypeStruct((N_NODE, D), jnp.float32),
        jax.ShapeDtypeStruct((N_EDGE, D), jnp.float32),
    ],
    scratch_shapes=[
        pltpu.VMEM((N_EDGE, D), jnp.float32),
        pltpu.VMEM((N_EDGE, 1), jnp.float32),
        pltpu.VMEM((1, D), jnp.float32),
    ],
)


def kernel(x, xe, pair, a, wt):
    m = _build_mask(pair[0], pair[1])
    xep, xa, xph, ts = _project(x, xe, wt, a.reshape(1, D))
    x_out, xe_out = _attend(xep, xa, xph, ts, m)
    return x_out, xe_out


# speedup vs baseline: 1.1285x; 1.0673x over previous
import functools

import jax
import jax.numpy as jnp
from jax import lax
from jax.experimental import pallas as pl
from jax.experimental.pallas import tpu as pltpu
from jax.experimental.pallas import tpu_sc as plsc

N_NODE = 10000
N_EDGE = 2000
N_PAIR = 160000
D = 128

NC = 2
NS = 16
ROWS_CORE = N_EDGE // NC
ROWS_P0 = 512
ROWS_P1 = ROWS_CORE - ROWS_P0
NBLK = -(-N_EDGE // 128)
UNIT = 8 * NBLK * 128
UNITS_P0 = ROWS_P0 // 8
UNITS_P1 = ROWS_P1 // 8
P0_WORDS = UNITS_P0 * UNIT
P1_WORDS = UNITS_P1 * UNIT
DUMP = P0_WORDS
PAIRS_MAIN = (N_PAIR // 128 // NS) * 128
TAIL_BASE = PAIRS_MAIN * NS
IDX_ROWS = PAIRS_MAIN // 128 + 1
ZCHUNK = 4096


def _mask_body(pair_hbm, out_hbm, shared, pairv, tailv, idxv,
               onesv, zbuf, semp, semz, sems, semo):
    cid = lax.axis_index("c")
    sid = lax.axis_index("s")

    hp = pltpu.async_copy(
        pair_hbm.at[:, pl.ds(sid * PAIRS_MAIN, PAIRS_MAIN)], pairv, semp)

    def _fill_z(i, _):
        zbuf[pl.ds(i * 16, 16)] = jnp.zeros((16,), jnp.float32)
        return 0

    with jax.named_scope("ph_fill"):
        lax.fori_loop(0, ZCHUNK // 16, _fill_z, 0)
        for k in range(8):
            onesv[pl.ds(k * 16, 16)] = jnp.ones((16,), jnp.float32)

    def _fire_zero(words):
        hs = []
        base = sid * words
        nfull, rem = divmod(words, ZCHUNK)
        for c in range(nfull):
            hs.append(pltpu.async_copy(
                zbuf, shared.at[pl.ds(base + c * ZCHUNK, ZCHUNK)], semz))
        if rem:
            hs.append(pltpu.async_copy(
                zbuf.at[pl.ds(0, rem)],
                shared.at[pl.ds(base + nfull * ZCHUNK, rem)], semz))
        return hs

    zh = _fire_zero(P0_WORDS // NS)

    with jax.named_scope("ph_pairwait"):
        hp.wait()

    @pl.when(sid < 2)
    def _():
        pltpu.sync_copy(pair_hbm.at[:, pl.ds(TAIL_BASE + sid * 128, 128)],
                        tailv)

    row_lo = cid * ROWS_CORE

    def _compute_idx(pass_lo, pass_rows):
        def _offs(p0, p1):
            r = p0 - pass_lo
            woff = (r << 11) + p1
            return jnp.where((r >= 0) & (r < pass_rows), woff, DUMP + p1)

        def _body(i, _):
            p0 = pairv[0, pl.ds(i * 16, 16)]
            p1 = pairv[1, pl.ds(i * 16, 16)]
            idxv[i // 8, pl.ds((i % 8) * 16, 16)] = _offs(p0, p1)
            return 0

        with jax.named_scope("ph_offsets"):
            lax.fori_loop(0, PAIRS_MAIN // 16, _body, 0)
            dump = jnp.full((16,), DUMP, jnp.int32)
            for k in range(8):
                idxv[IDX_ROWS - 1, pl.ds(k * 16, 16)] = dump

            @pl.when(sid < 2)
            def _():
                for k in range(8):
                    p0 = tailv[0, pl.ds(k * 16, 16)]
                    p1 = tailv[1, pl.ds(k * 16, 16)]
                    idxv[IDX_ROWS - 1, pl.ds(k * 16, 16)] = _offs(p0, p1)

    def _scatter():
        with jax.named_scope("ph_scatter"):
            sh = [pltpu.async_copy(onesv, shared.at[idxv.at[j]], sems)
                  for j in range(IDX_ROWS)]
            for h in sh:
                h.wait()

    def _copy_out(units, unit0):
        with jax.named_scope("ph_copyout"):
            return [pltpu.async_copy(
                shared.at[pl.ds(u * UNIT, UNIT)],
                out_hbm.at[pl.ds((cid * (ROWS_CORE // 8) + unit0 + u) * UNIT,
                                 UNIT)], semo)
                for u in units]

    _compute_idx(row_lo, ROWS_P0)
    with jax.named_scope("ph_zerowait"):
        for h in zh:
            h.wait()
    plsc.subcore_barrier()
    _scatter()
    plsc.subcore_barrier()
    upt = UNITS_P0 // NS
    pend = _copy_out([sid * upt + k for k in range(upt)], 0)
    for h in pend:
        h.wait()
    plsc.subcore_barrier()

    zh = _fire_zero(P1_WORDS // NS)
    _compute_idx(row_lo + ROWS_P0, ROWS_P1)
    with jax.named_scope("ph_zerowait"):
        for h in zh:
            h.wait()
    plsc.subcore_barrier()
    _scatter()
    plsc.subcore_barrier()
    pend = _copy_out([sid + NS * k for k in range(UNITS_P1 // NS)], UNITS_P0)
    for h in pend:
        h.wait()

    @pl.when(sid < UNITS_P1 % NS)
    def _():
        u = sid + NS * (UNITS_P1 // NS)
        pltpu.sync_copy(
            shared.at[pl.ds(u * UNIT, UNIT)],
            out_hbm.at[pl.ds((cid * (ROWS_CORE // 8) + UNITS_P0 + u) * UNIT,
                             UNIT)])


_build_mask = functools.partial(
    pl.kernel,
    mesh=plsc.VectorSubcoreMesh(core_axis_name="c", subcore_axis_name="s"),
    out_type=jax.ShapeDtypeStruct(((N_EDGE // 8) * UNIT,), jnp.float32),
    scratch_types=[
        pltpu.VMEM_SHARED((P0_WORDS + 2048,), jnp.float32),
        pltpu.VMEM((2, PAIRS_MAIN), jnp.int32),
        pltpu.VMEM((2, 128), jnp.int32),
        pltpu.VMEM((IDX_ROWS, 128), jnp.int32),
        pltpu.VMEM((128,), jnp.float32),
        pltpu.VMEM((ZCHUNK,), jnp.float32),
        pltpu.SemaphoreType.DMA,
        pltpu.SemaphoreType.DMA,
        pltpu.SemaphoreType.DMA,
        pltpu.SemaphoreType.DMA,
    ],
)(_mask_body)


def _proj_body(x_ref, xe_ref, wt_ref, a_ref, xep_ref, xa_ref, xph_ref, ts_ref):
    wt = wt_ref[...]
    xp = jnp.dot(x_ref[...], wt, preferred_element_type=jnp.float32)
    xep_ref[...] = jnp.dot(xe_ref[...], wt, preferred_element_type=jnp.float32)
    xph = xp[:N_EDGE]
    xph_ref[...] = xph
    xa_ref[...] = xph * a_ref[...]
    ts_ref[...] = jnp.sum(xp[N_EDGE:], axis=0, keepdims=True)


_project = pl.pallas_call(
    _proj_body,
    out_shape=[
        jax.ShapeDtypeStruct((N_EDGE, D), jnp.float32),
        jax.ShapeDtypeStruct((N_EDGE, D), jnp.float32),
        jax.ShapeDtypeStruct((N_EDGE, D), jnp.float32),
        jax.ShapeDtypeStruct((1, D), jnp.float32),
    ],
)


BR = 400
NSTEPS = N_EDGE // BR


def _attn_body(xep_ref, xa_ref, xph_ref, ts_ref, m_ref,
               xout_ref, xeout_ref, nacc, zacc, xsacc):
    i = pl.program_id(0)
    xep = xep_ref[...]
    s = lax.dot_general(xep, xa_ref[...], (((1,), (1,)), ((), ())),
                        preferred_element_type=jnp.float32)
    m = jnp.reshape(m_ref[...], (BR, 16 * 128))[:, :N_EDGE]
    g = 1e-10 + jnp.where(m > 0, jnp.exp(s), 0.0)

    ze = jnp.sum(g, axis=1, keepdims=True) + (N_NODE - N_EDGE) * 1e-10
    pe = g / ze
    xe_out = (jnp.dot(pe, xph_ref[...], preferred_element_type=jnp.float32)
              + (1e-10 / ze) * ts_ref[...])
    xeout_ref[...] = jnp.where(xe_out > 0, xe_out, jnp.exp(xe_out) - 1.0)

    @pl.when(i == 0)
    def _():
        nacc[...] = jnp.zeros((N_EDGE, D), jnp.float32)
        zacc[...] = jnp.zeros((N_EDGE, 1), jnp.float32)
        xsacc[...] = jnp.zeros((1, D), jnp.float32)

    nacc[...] += lax.dot_general(g, xep, (((0,), (0,)), ((), ())),
                                 preferred_element_type=jnp.float32)
    zacc[...] += lax.dot_general(g, jnp.ones((BR, 1), jnp.float32),
                                 (((0,), (0,)), ((), ())),
                                 preferred_element_type=jnp.float32)
    xsacc[...] += jnp.sum(xep, axis=0, keepdims=True)

    @pl.when(i == NSTEPS - 1)
    def _():
        x_head = nacc[...] / zacc[...]
        xout_ref[:N_EDGE] = jnp.where(x_head > 0, x_head,
                                      jnp.exp(x_head) - 1.0)
        x_tail = xsacc[...] * (1.0 / N_EDGE)
        x_tail = jnp.where(x_tail > 0, x_tail, jnp.exp(x_tail) - 1.0)
        xout_ref[N_EDGE:] = jnp.broadcast_to(x_tail, (N_NODE - N_EDGE, D))


_attend = pl.pallas_call(
    _attn_body,
    grid=(NSTEPS,),
    in_specs=[
        pl.BlockSpec((BR, D), lambda i: (i, 0)),
        pl.BlockSpec((N_EDGE, D), lambda i: (0, 0)),
        pl.BlockSpec((N_EDGE, D), lambda i: (0, 0)),
        pl.BlockSpec((1, D), lambda i: (0, 0)),
        pl.BlockSpec((BR * 16 * 128,), lambda i: (i,)),
    ],
    out_specs=[
        pl.BlockSpec((N_NODE, D), lambda i: (0, 0)),
        pl.BlockSpec((BR, D), lambda i: (i, 0)),
    ],
    out_shape=[
        jax.ShapeDtypeStruct((N_NODE, D), jnp.float32),
        jax.ShapeDtypeStruct((N_EDGE, D), jnp.float32),
    ],
    scratch_shapes=[
        pltpu.VMEM((N_EDGE, D), jnp.float32),
        pltpu.VMEM((N_EDGE, 1), jnp.float32),
        pltpu.VMEM((1, D), jnp.float32),
    ],
)


def kernel(x, xe, pair, a, wt):
    m = _build_mask(pair)
    xep, xa, xph, ts = _project(x, xe, wt, a.reshape(1, D))
    x_out, xe_out = _attend(xep, xa, xph, ts, m)
    return x_out, xe_out


# speedup vs baseline: 1.1598x; 1.0277x over previous
import functools

import jax
import jax.numpy as jnp
from jax import lax
from jax.experimental import pallas as pl
from jax.experimental.pallas import tpu as pltpu
from jax.experimental.pallas import tpu_sc as plsc

N_NODE = 10000
N_EDGE = 2000
N_PAIR = 160000
D = 128

NC = 2
NS = 16
ROWS_CORE = N_EDGE // NC
ROWS_P0 = 512
ROWS_P1 = ROWS_CORE - ROWS_P0
NBLK = -(-N_EDGE // 128)
UNIT = 8 * NBLK * 128
UNITS_P0 = ROWS_P0 // 8
UNITS_P1 = ROWS_P1 // 8
P0_WORDS = UNITS_P0 * UNIT
P1_WORDS = UNITS_P1 * UNIT
DUMP = P0_WORDS
PAIRS_MAIN = (N_PAIR // 128 // NS) * 128
TAIL_BASE = PAIRS_MAIN * NS
IDX_ROWS = PAIRS_MAIN // 128 + 1
ZCHUNK = 4096


def _mask_body(pair_hbm, out_hbm, shared, pairv, tailv, idxv,
               onesv, zbuf, semp, semz, sems, semo):
    cid = lax.axis_index("c")
    sid = lax.axis_index("s")

    hp = pltpu.async_copy(
        pair_hbm.at[:, pl.ds(sid * PAIRS_MAIN, PAIRS_MAIN)], pairv, semp)

    def _fill_z(i, _):
        zbuf[pl.ds(i * 16, 16)] = jnp.zeros((16,), jnp.float32)
        return 0

    with jax.named_scope("ph_fill"):
        lax.fori_loop(0, ZCHUNK // 16, _fill_z, 0)
        for k in range(8):
            onesv[pl.ds(k * 16, 16)] = jnp.ones((16,), jnp.float32)

    def _fire_zero(words):
        hs = []
        base = sid * words
        nfull, rem = divmod(words, ZCHUNK)
        for c in range(nfull):
            hs.append(pltpu.async_copy(
                zbuf, shared.at[pl.ds(base + c * ZCHUNK, ZCHUNK)], semz))
        if rem:
            hs.append(pltpu.async_copy(
                zbuf.at[pl.ds(0, rem)],
                shared.at[pl.ds(base + nfull * ZCHUNK, rem)], semz))
        return hs

    zh = _fire_zero(P0_WORDS // NS)

    with jax.named_scope("ph_pairwait"):
        hp.wait()

    @pl.when(sid < 2)
    def _():
        pltpu.sync_copy(pair_hbm.at[:, pl.ds(TAIL_BASE + sid * 128, 128)],
                        tailv)

    row_lo = cid * ROWS_CORE

    def _compute_idx(pass_lo, pass_rows):
        def _offs(p0, p1):
            r = p0 - pass_lo
            woff = (r << 11) + p1
            return jnp.where((r >= 0) & (r < pass_rows), woff, DUMP + p1)

        def _body(i, _):
            p0 = pairv[0, pl.ds(i * 16, 16)]
            p1 = pairv[1, pl.ds(i * 16, 16)]
            idxv[i // 8, pl.ds((i % 8) * 16, 16)] = _offs(p0, p1)
            return 0

        with jax.named_scope("ph_offsets"):
            lax.fori_loop(0, PAIRS_MAIN // 16, _body, 0)
            dump = jnp.full((16,), DUMP, jnp.int32)
            for k in range(8):
                idxv[IDX_ROWS - 1, pl.ds(k * 16, 16)] = dump

            @pl.when(sid < 2)
            def _():
                for k in range(8):
                    p0 = tailv[0, pl.ds(k * 16, 16)]
                    p1 = tailv[1, pl.ds(k * 16, 16)]
                    idxv[IDX_ROWS - 1, pl.ds(k * 16, 16)] = _offs(p0, p1)

    def _scatter():
        with jax.named_scope("ph_scatter"):
            sh = [pltpu.async_copy(onesv, shared.at[idxv.at[j]], sems)
                  for j in range(IDX_ROWS)]
            for h in sh:
                h.wait()

    def _copy_out(units, unit0):
        with jax.named_scope("ph_copyout"):
            return [pltpu.async_copy(
                shared.at[pl.ds(u * UNIT, UNIT)],
                out_hbm.at[pl.ds((cid * (ROWS_CORE // 8) + unit0 + u) * UNIT,
                                 UNIT)], semo)
                for u in units]

    _compute_idx(row_lo, ROWS_P0)
    with jax.named_scope("ph_zerowait"):
        for h in zh:
            h.wait()
    plsc.subcore_barrier()
    _scatter()
    plsc.subcore_barrier()
    upt = UNITS_P0 // NS
    pend = _copy_out([sid * upt + k for k in range(upt)], 0)
    for h in pend:
        h.wait()
    plsc.subcore_barrier()

    zh = _fire_zero(P1_WORDS // NS)
    _compute_idx(row_lo + ROWS_P0, ROWS_P1)
    with jax.named_scope("ph_zerowait"):
        for h in zh:
            h.wait()
    plsc.subcore_barrier()
    _scatter()
    plsc.subcore_barrier()
    pend = _copy_out([sid + NS * k for k in range(UNITS_P1 // NS)], UNITS_P0)
    for h in pend:
        h.wait()

    @pl.when(sid < UNITS_P1 % NS)
    def _():
        u = sid + NS * (UNITS_P1 // NS)
        pltpu.sync_copy(
            shared.at[pl.ds(u * UNIT, UNIT)],
            out_hbm.at[pl.ds((cid * (ROWS_CORE // 8) + UNITS_P0 + u) * UNIT,
                             UNIT)])


_build_mask = functools.partial(
    pl.kernel,
    mesh=plsc.VectorSubcoreMesh(core_axis_name="c", subcore_axis_name="s"),
    out_type=jax.ShapeDtypeStruct(((N_EDGE // 8) * UNIT,), jnp.float32),
    scratch_types=[
        pltpu.VMEM_SHARED((P0_WORDS + 2048,), jnp.float32),
        pltpu.VMEM((2, PAIRS_MAIN), jnp.int32),
        pltpu.VMEM((2, 128), jnp.int32),
        pltpu.VMEM((IDX_ROWS, 128), jnp.int32),
        pltpu.VMEM((128,), jnp.float32),
        pltpu.VMEM((ZCHUNK,), jnp.float32),
        pltpu.SemaphoreType.DMA,
        pltpu.SemaphoreType.DMA,
        pltpu.SemaphoreType.DMA,
        pltpu.SemaphoreType.DMA,
    ],
)(_mask_body)


def _proj_body(x_ref, xe_ref, wt_ref, a_ref, xep_ref, xa_ref, xph_ref, ts_ref):
    wt = wt_ref[...]
    xp = jnp.dot(x_ref[...], wt, preferred_element_type=jnp.float32)
    xep_ref[...] = jnp.dot(xe_ref[...], wt, preferred_element_type=jnp.float32)
    xph = xp[:N_EDGE]
    xph_ref[...] = xph
    xa_ref[...] = xph * a_ref[...]
    ts_ref[...] = jnp.sum(xp[N_EDGE:], axis=0, keepdims=True)


_project = pl.pallas_call(
    _proj_body,
    out_shape=[
        jax.ShapeDtypeStruct((N_EDGE, D), jnp.float32),
        jax.ShapeDtypeStruct((N_EDGE, D), jnp.float32),
        jax.ShapeDtypeStruct((N_EDGE, D), jnp.float32),
        jax.ShapeDtypeStruct((1, D), jnp.float32),
    ],
)


BR = 1000
NSTEPS = N_EDGE // BR


def _attn_body(xep_ref, xa_ref, xph_ref, ts_ref, m_ref,
               xout_ref, xeout_ref, nacc, zacc, xsacc):
    i = pl.program_id(0)
    xep = xep_ref[...]
    s = lax.dot_general(xep, xa_ref[...], (((1,), (1,)), ((), ())),
                        preferred_element_type=jnp.float32)
    m = jnp.reshape(m_ref[...], (BR, 16 * 128))[:, :N_EDGE]
    g = 1e-10 + jnp.where(m > 0, jnp.exp(s), 0.0)

    ze = jnp.sum(g, axis=1, keepdims=True) + (N_NODE - N_EDGE) * 1e-10
    pe = g / ze
    xe_out = (jnp.dot(pe, xph_ref[...], preferred_element_type=jnp.float32)
              + (1e-10 / ze) * ts_ref[...])
    xeout_ref[...] = jnp.where(xe_out > 0, xe_out, jnp.exp(xe_out) - 1.0)

    @pl.when(i == 0)
    def _():
        nacc[...] = jnp.zeros((N_EDGE, D), jnp.float32)
        zacc[...] = jnp.zeros((N_EDGE, 1), jnp.float32)
        xsacc[...] = jnp.zeros((1, D), jnp.float32)

    nacc[...] += lax.dot_general(g, xep, (((0,), (0,)), ((), ())),
                                 preferred_element_type=jnp.float32)
    zacc[...] += lax.dot_general(g, jnp.ones((BR, 1), jnp.float32),
                                 (((0,), (0,)), ((), ())),
                                 preferred_element_type=jnp.float32)
    xsacc[...] += jnp.sum(xep, axis=0, keepdims=True)

    @pl.when(i == NSTEPS - 1)
    def _():
        x_head = nacc[...] / zacc[...]
        xout_ref[:N_EDGE] = jnp.where(x_head > 0, x_head,
                                      jnp.exp(x_head) - 1.0)
        x_tail = xsacc[...] * (1.0 / N_EDGE)
        x_tail = jnp.where(x_tail > 0, x_tail, jnp.exp(x_tail) - 1.0)
        xout_ref[N_EDGE:] = jnp.broadcast_to(x_tail, (N_NODE - N_EDGE, D))


_attend = pl.pallas_call(
    _attn_body,
    grid=(NSTEPS,),
    in_specs=[
        pl.BlockSpec((BR, D), lambda i: (i, 0)),
        pl.BlockSpec((N_EDGE, D), lambda i: (0, 0)),
        pl.BlockSpec((N_EDGE, D), lambda i: (0, 0)),
        pl.BlockSpec((1, D), lambda i: (0, 0)),
        pl.BlockSpec((BR * 16 * 128,), lambda i: (i,)),
    ],
    out_specs=[
        pl.BlockSpec((N_NODE, D), lambda i: (0, 0)),
        pl.BlockSpec((BR, D), lambda i: (i, 0)),
    ],
    out_shape=[
        jax.ShapeDtypeStruct((N_NODE, D), jnp.float32),
        jax.ShapeDtypeStruct((N_EDGE, D), jnp.float32),
    ],
    scratch_shapes=[
        pltpu.VMEM((N_EDGE, D), jnp.float32),
        pltpu.VMEM((N_EDGE, 1), jnp.float32),
        pltpu.VMEM((1, D), jnp.float32),
    ],
)


def kernel(x, xe, pair, a, wt):
    m = _build_mask(pair)
    xep, xa, xph, ts = _project(x, xe, wt, a.reshape(1, D))
    x_out, xe_out = _attend(xep, xa, xph, ts, m)
    return x_out, xe_out
